# baseline (speedup 1.0000x reference)
import functools

import numpy as np

import jax
import jax.numpy as jnp
from jax import lax
from jax.experimental import pallas as pl
from jax.experimental.pallas import tpu as pltpu
from jax.experimental.pallas import tpu_sc as plsc

BATCH = 16384
NUM_FEATURES = 10
NUM_VALUES = 500
HIDDEN = 128

NUM_CORES = 2
NUM_SUBCORES = 16
NUM_WORKERS = NUM_CORES * NUM_SUBCORES
B_PER_W = BATCH // NUM_WORKERS
CHUNK = 128
N_CHUNKS = B_PER_W // CHUNK
LANES = 16
VECS_PER_ROW = HIDDEN // LANES
GROUPS_PER_CHUNK = CHUNK // LANES
WINDOW = LANES * NUM_FEATURES

TAB_ROWS = NUM_FEATURES * NUM_VALUES
STAGE_ROWS = TAB_ROWS // NUM_SUBCORES
STAGE_TAIL = TAB_ROWS - (NUM_SUBCORES - 1) * STAGE_ROWS

_GATHER_DNUMS = lax.GatherDimensionNumbers(
    offset_dims=(), collapsed_slice_dims=(0,), start_index_map=(0,))

_PERM = []
for _f in range(NUM_FEATURES):
    _e = 10 * np.arange(LANES) + _f
    _ks = _e // LANES
    _per_f = []
    for _k in sorted(set(_ks.tolist())):
        _m = _ks == _k
        _idx = np.where(_m, _e - LANES * _k, 0).astype(np.int32)
        _per_f.append((_k, _idx, _m))
    _PERM.append(_per_f)


def _sc_encode(xf, tab):
    mesh = plsc.VectorSubcoreMesh(core_axis_name="c", subcore_axis_name="s")

    @functools.partial(
        pl.kernel,
        mesh=mesh,
        out_type=jax.ShapeDtypeStruct((BATCH, HIDDEN), jnp.float32),
        scratch_types=[
            pltpu.VMEM((B_PER_W * NUM_FEATURES,), jnp.int32),
            pltpu.VMEM((NUM_FEATURES, N_CHUNKS, CHUNK), jnp.int32),
            pltpu.VMEM((2, CHUNK, HIDDEN), jnp.float32),
            pltpu.VMEM_SHARED((TAB_ROWS, HIDDEN), jnp.float32),
            pltpu.SemaphoreType.DMA,
            pltpu.SemaphoreType.DMA,
            pltpu.SemaphoreType.DMA((2,)),
            pltpu.SemaphoreType.DMA((2,)),
        ],
    )
    def k(xf_hbm, tab_hbm, out_hbm, xsl, idx_all, acc2, shared_tab,
          ssem, hsem, gsem, osem):
        wid = lax.axis_index("s") * NUM_CORES + lax.axis_index("c")
        sid = lax.axis_index("s")
        base = wid * B_PER_W

        @pl.when(sid < NUM_SUBCORES - 1)
        def _():
            pltpu.async_copy(
                tab_hbm.at[pl.ds(sid * STAGE_ROWS, STAGE_ROWS)],
                shared_tab.at[pl.ds(sid * STAGE_ROWS, STAGE_ROWS)],
                ssem,
            )

        @pl.when(sid == NUM_SUBCORES - 1)
        def _():
            pltpu.async_copy(
                tab_hbm.at[pl.ds((NUM_SUBCORES - 1) * STAGE_ROWS, STAGE_TAIL)],
                shared_tab.at[
                    pl.ds((NUM_SUBCORES - 1) * STAGE_ROWS, STAGE_TAIL)],
                ssem,
            )

        pltpu.sync_copy(
            xf_hbm.at[pl.ds(base * NUM_FEATURES, B_PER_W * NUM_FEATURES)],
            xsl)

        lane_ii = lax.iota(jnp.int32, LANES)
        perm = []
        for f in range(NUM_FEATURES):
            e = lane_ii * NUM_FEATURES + f
            per_f = []
            for kk, _idx_np, _m_np in _PERM[f]:
                d = e - kk * LANES
                m = (d >= 0) & (d < LANES)
                iv = jnp.clip(d, 0, LANES - 1)[:, None]
                per_f.append((kk, iv, m))
            perm.append(per_f)

        def build_idx(cc):
            def wnd(jj, _):
                wbase = pl.multiple_of((cc * GROUPS_PER_CHUNK + jj) * WINDOW,
                                       LANES)
                vs = [xsl[pl.ds(wbase + kk * LANES, LANES)]
                      for kk in range(NUM_FEATURES)]
                for f in range(NUM_FEATURES):
                    r = None
                    for kk, iv, m in perm[f]:
                        p = lax.gather(
                            vs[kk], iv,
                            _GATHER_DNUMS, slice_sizes=(1,),
                            mode=lax.GatherScatterMode.PROMISE_IN_BOUNDS)
                        if r is None:
                            r = p
                        else:
                            r = jnp.where(m, p, r)
                    idx_all[f, cc, pl.ds(jj * LANES, LANES)] = (
                        r + f * NUM_VALUES)
                return 0

            lax.fori_loop(0, GROUPS_PER_CHUNK, wnd, 0)

        zero16 = jnp.zeros((LANES,), jnp.float32)

        def zero_acc(b):
            def zrow(i, _):
                for j in range(VECS_PER_ROW):
                    acc2.at[b][i, pl.ds(j * LANES, LANES)] = zero16
                return 0

            lax.fori_loop(0, CHUNK, zrow, 0)

        def fire_hbm_gathers(cc, b):
            def feat(f, _):
                pltpu.async_copy(
                    tab_hbm.at[idx_all.at[f, cc]], acc2.at[b], hsem,
                    add=True,
                )
                return 0

            lax.fori_loop(0, NUM_FEATURES, feat, 0)

        def fire_gathers(cc, b):
            def feat(f, _):
                pltpu.async_copy(
                    shared_tab.at[idx_all.at[f, cc]], acc2.at[b], gsem.at[b],
                    add=True,
                )
                return 0

            lax.fori_loop(0, NUM_FEATURES, feat, 0)

        def drain_gathers(cc, b):
            def feat_h(f, _):
                pltpu.make_async_copy(
                    tab_hbm.at[idx_all.at[f, cc]], acc2.at[b], hsem
                ).wait()
                return 0

            def feat_s(f, _):
                pltpu.make_async_copy(
                    shared_tab.at[idx_all.at[f, cc]], acc2.at[b], gsem.at[b]
                ).wait()
                return 0

            @pl.when(cc == 0)
            def _():
                lax.fori_loop(0, NUM_FEATURES, feat_h, 0)

            @pl.when(cc > 0)
            def _():
                lax.fori_loop(0, NUM_FEATURES, feat_s, 0)

        build_idx(0)
        zero_acc(0)
        fire_hbm_gathers(0, 0)

        @pl.when(sid < NUM_SUBCORES - 1)
        def _():
            pltpu.make_async_copy(
                tab_hbm.at[pl.ds(sid * STAGE_ROWS, STAGE_ROWS)],
                shared_tab.at[pl.ds(sid * STAGE_ROWS, STAGE_ROWS)],
                ssem,
            ).wait()

        @pl.when(sid == NUM_SUBCORES - 1)
        def _():
            pltpu.make_async_copy(
                tab_hbm.at[pl.ds((NUM_SUBCORES - 1) * STAGE_ROWS, STAGE_TAIL)],
                shared_tab.at[
                    pl.ds((NUM_SUBCORES - 1) * STAGE_ROWS, STAGE_TAIL)],
                ssem,
            ).wait()

        plsc.subcore_barrier()

        def chunk_body(c, _):
            p = c % 2
            q = 1 - p

            @pl.when(c < N_CHUNKS - 1)
            def _prep_next():
                @pl.when(c >= 1)
                def _():
                    pltpu.make_async_copy(
                        acc2.at[q],
                        out_hbm.at[pl.ds(base + (c - 1) * CHUNK, CHUNK)],
                        osem.at[q],
                    ).wait()

                build_idx(c + 1)
                zero_acc(q)
                fire_gathers(c + 1, q)

            drain_gathers(c, p)
            pltpu.async_copy(
                acc2.at[p], out_hbm.at[pl.ds(base + c * CHUNK, CHUNK)],
                osem.at[p],
            )
            return 0

        lax.fori_loop(0, N_CHUNKS, chunk_body, 0)

        for c in (N_CHUNKS - 2, N_CHUNKS - 1):
            pltpu.make_async_copy(
                acc2.at[c % 2],
                out_hbm.at[pl.ds(base + c * CHUNK, CHUNK)],
                osem.at[c % 2],
            ).wait()

    return k(xf, tab)


def kernel(x, tables):
    if x.ndim == 1:
        x = x[:, None]
    return _sc_encode(x.astype(jnp.int32).reshape(BATCH * NUM_FEATURES),
                      tables.reshape(TAB_ROWS, HIDDEN))


# speedup vs baseline: 1.2698x; 1.2698x over previous
import functools

import jax
import jax.numpy as jnp
from jax import lax
from jax.experimental import pallas as pl
from jax.experimental.pallas import tpu as pltpu
from jax.experimental.pallas import tpu_sc as plsc

BATCH = 16384
NUM_FEATURES = 10
NUM_VALUES = 500
HIDDEN = 128

NUM_CORES = 2
NUM_SUBCORES = 16
NUM_WORKERS = NUM_CORES * NUM_SUBCORES
B_PER_W = BATCH // NUM_WORKERS
CHUNK = 64
N_CHUNKS = B_PER_W // CHUNK
LANES = 16
VECS_PER_ROW = HIDDEN // LANES
GROUPS_PER_CHUNK = CHUNK // LANES

TAB_ROWS = NUM_FEATURES * NUM_VALUES
STAGE_ROWS = TAB_ROWS // NUM_SUBCORES
STAGE_TAIL = TAB_ROWS - (NUM_SUBCORES - 1) * STAGE_ROWS


def _sc_encode(xi, tab):
    mesh = plsc.VectorSubcoreMesh(core_axis_name="c", subcore_axis_name="s")

    @functools.partial(
        pl.kernel,
        mesh=mesh,
        out_type=jax.ShapeDtypeStruct((BATCH, HIDDEN), jnp.float32),
        scratch_types=[
            pltpu.VMEM((NUM_FEATURES, N_CHUNKS, CHUNK), jnp.int32),
            pltpu.VMEM((2, CHUNK, HIDDEN), jnp.float32),
            pltpu.VMEM_SHARED((TAB_ROWS, HIDDEN), jnp.float32),
            pltpu.SemaphoreType.DMA,
            pltpu.SemaphoreType.DMA,
            pltpu.SemaphoreType.DMA((2,)),
            pltpu.SemaphoreType.DMA((2,)),
        ],
    )
    def k(xi_hbm, tab_hbm, out_hbm, idx_all, acc2, shared_tab,
          ssem, hsem, gsem, osem):
        wid = lax.axis_index("s") * NUM_CORES + lax.axis_index("c")
        sid = lax.axis_index("s")
        base = wid * B_PER_W

        @pl.when(sid < NUM_SUBCORES - 1)
        def _():
            pltpu.async_copy(
                tab_hbm.at[pl.ds(sid * STAGE_ROWS, STAGE_ROWS)],
                shared_tab.at[pl.ds(sid * STAGE_ROWS, STAGE_ROWS)],
                ssem,
            )

        @pl.when(sid == NUM_SUBCORES - 1)
        def _():
            pltpu.async_copy(
                tab_hbm.at[pl.ds((NUM_SUBCORES - 1) * STAGE_ROWS, STAGE_TAIL)],
                shared_tab.at[
                    pl.ds((NUM_SUBCORES - 1) * STAGE_ROWS, STAGE_TAIL)],
                ssem,
            )

        pltpu.sync_copy(xi_hbm.at[wid], idx_all)

        zero16 = jnp.zeros((LANES,), jnp.float32)

        def zero_acc(b):
            def zrow(i, _):
                for j in range(VECS_PER_ROW):
                    acc2.at[b][i, pl.ds(j * LANES, LANES)] = zero16
                return 0

            lax.fori_loop(0, CHUNK, zrow, 0)

        def fire_hbm_gathers(cc, b):
            def feat(f, _):
                pltpu.async_copy(
                    tab_hbm.at[idx_all.at[f, cc]], acc2.at[b], hsem,
                    add=True,
                )
                return 0

            lax.fori_loop(0, NUM_FEATURES, feat, 0)

        def fire_gathers(cc, b):
            def feat(f, _):
                pltpu.async_copy(
                    shared_tab.at[idx_all.at[f, cc]], acc2.at[b], gsem.at[b],
                    add=True,
                )
                return 0

            lax.fori_loop(0, NUM_FEATURES, feat, 0)

        def drain_gathers(cc, b):
            def feat_h(f, _):
                pltpu.make_async_copy(
                    tab_hbm.at[idx_all.at[f, cc]], acc2.at[b], hsem
                ).wait()
                return 0

            def feat_s(f, _):
                pltpu.make_async_copy(
                    shared_tab.at[idx_all.at[f, cc]], acc2.at[b], gsem.at[b]
                ).wait()
                return 0

            @pl.when(cc == 0)
            def _():
                lax.fori_loop(0, NUM_FEATURES, feat_h, 0)

            @pl.when(cc > 0)
            def _():
                lax.fori_loop(0, NUM_FEATURES, feat_s, 0)

        zero_acc(0)
        fire_hbm_gathers(0, 0)

        @pl.when(sid < NUM_SUBCORES - 1)
        def _():
            pltpu.make_async_copy(
                tab_hbm.at[pl.ds(sid * STAGE_ROWS, STAGE_ROWS)],
                shared_tab.at[pl.ds(sid * STAGE_ROWS, STAGE_ROWS)],
                ssem,
            ).wait()

        @pl.when(sid == NUM_SUBCORES - 1)
        def _():
            pltpu.make_async_copy(
                tab_hbm.at[pl.ds((NUM_SUBCORES - 1) * STAGE_ROWS, STAGE_TAIL)],
                shared_tab.at[
                    pl.ds((NUM_SUBCORES - 1) * STAGE_ROWS, STAGE_TAIL)],
                ssem,
            ).wait()

        plsc.subcore_barrier()

        def chunk_body(c, _):
            p = c % 2
            q = 1 - p

            @pl.when(c < N_CHUNKS - 1)
            def _prep_next():
                @pl.when(c >= 1)
                def _():
                    pltpu.make_async_copy(
                        acc2.at[q],
                        out_hbm.at[pl.ds(base + (c - 1) * CHUNK, CHUNK)],
                        osem.at[q],
                    ).wait()

                zero_acc(q)
                fire_gathers(c + 1, q)

            drain_gathers(c, p)
            pltpu.async_copy(
                acc2.at[p], out_hbm.at[pl.ds(base + c * CHUNK, CHUNK)],
                osem.at[p],
            )
            return 0

        lax.fori_loop(0, N_CHUNKS, chunk_body, 0)

        for c in (N_CHUNKS - 2, N_CHUNKS - 1):
            pltpu.make_async_copy(
                acc2.at[c % 2],
                out_hbm.at[pl.ds(base + c * CHUNK, CHUNK)],
                osem.at[c % 2],
            ).wait()

    return k(xi, tab)


def kernel(x, tables):
    if x.ndim == 1:
        x = x[:, None]
    xi = x.astype(jnp.int32) + NUM_VALUES * jnp.arange(
        NUM_FEATURES, dtype=jnp.int32)[None, :]
    xi = xi.reshape(NUM_WORKERS, N_CHUNKS, CHUNK, NUM_FEATURES)
    xi = xi.transpose(0, 3, 1, 2)
    return _sc_encode(xi, tables.reshape(TAB_ROWS, HIDDEN))


# speedup vs baseline: 1.3406x; 1.0558x over previous
import functools

import jax
import jax.numpy as jnp
from jax import lax
from jax.experimental import pallas as pl
from jax.experimental.pallas import tpu as pltpu
from jax.experimental.pallas import tpu_sc as plsc

BATCH = 16384
NUM_FEATURES = 10
NUM_VALUES = 500
HIDDEN = 128

NUM_CORES = 2
NUM_SUBCORES = 16
NUM_WORKERS = NUM_CORES * NUM_SUBCORES
B_PER_W = BATCH // NUM_WORKERS
CHUNK = 128
N_CHUNKS = B_PER_W // CHUNK
LANES = 16
VECS_PER_ROW = HIDDEN // LANES
GROUPS_PER_CHUNK = CHUNK // LANES

TAB_ROWS = NUM_FEATURES * NUM_VALUES
STAGE_ROWS = TAB_ROWS // NUM_SUBCORES
STAGE_TAIL = TAB_ROWS - (NUM_SUBCORES - 1) * STAGE_ROWS


def _sc_encode(xi, tab):
    mesh = plsc.VectorSubcoreMesh(core_axis_name="c", subcore_axis_name="s")

    @functools.partial(
        pl.kernel,
        mesh=mesh,
        out_type=jax.ShapeDtypeStruct((BATCH, HIDDEN), jnp.float32),
        scratch_types=[
            pltpu.VMEM((NUM_FEATURES, N_CHUNKS, CHUNK), jnp.int32),
            pltpu.VMEM((3, CHUNK, HIDDEN), jnp.float32),
            pltpu.VMEM_SHARED((TAB_ROWS, HIDDEN), jnp.float32),
            pltpu.SemaphoreType.DMA,
            pltpu.SemaphoreType.DMA,
            pltpu.SemaphoreType.DMA((3,)),
            pltpu.SemaphoreType.DMA((3,)),
        ],
    )
    def k(xi_hbm, tab_hbm, out_hbm, idx_all, acc2, shared_tab,
          ssem, hsem, gsem, osem):
        wid = lax.axis_index("s") * NUM_CORES + lax.axis_index("c")
        sid = lax.axis_index("s")
        base = wid * B_PER_W

        @pl.when(sid < NUM_SUBCORES - 1)
        def _():
            pltpu.async_copy(
                tab_hbm.at[pl.ds(sid * STAGE_ROWS, STAGE_ROWS)],
                shared_tab.at[pl.ds(sid * STAGE_ROWS, STAGE_ROWS)],
                ssem,
            )

        @pl.when(sid == NUM_SUBCORES - 1)
        def _():
            pltpu.async_copy(
                tab_hbm.at[pl.ds((NUM_SUBCORES - 1) * STAGE_ROWS, STAGE_TAIL)],
                shared_tab.at[
                    pl.ds((NUM_SUBCORES - 1) * STAGE_ROWS, STAGE_TAIL)],
                ssem,
            )

        pltpu.sync_copy(xi_hbm.at[wid], idx_all)

        zero16 = jnp.zeros((LANES,), jnp.float32)

        def zero_acc(b):
            def zrow(i, _):
                for j in range(VECS_PER_ROW):
                    acc2.at[b][i, pl.ds(j * LANES, LANES)] = zero16
                return 0

            lax.fori_loop(0, CHUNK, zrow, 0)

        def fire_hbm_gathers(cc, b):
            def feat(f, _):
                pltpu.async_copy(
                    tab_hbm.at[idx_all.at[f, cc]], acc2.at[b], hsem,
                    add=True,
                )
                return 0

            lax.fori_loop(0, NUM_FEATURES, feat, 0)

        def fire_gathers(cc, b):
            def feat(f, _):
                pltpu.async_copy(
                    shared_tab.at[idx_all.at[f, cc]], acc2.at[b], gsem.at[b],
                    add=True,
                )
                return 0

            lax.fori_loop(0, NUM_FEATURES, feat, 0)

        def drain_gathers(cc, b):
            def feat_h(f, _):
                pltpu.make_async_copy(
                    tab_hbm.at[idx_all.at[f, cc]], acc2.at[b], hsem
                ).wait()
                return 0

            def feat_s(f, _):
                pltpu.make_async_copy(
                    shared_tab.at[idx_all.at[f, cc]], acc2.at[b], gsem.at[b]
                ).wait()
                return 0

            @pl.when(cc == 0)
            def _():
                lax.fori_loop(0, NUM_FEATURES, feat_h, 0)

            @pl.when(cc > 0)
            def _():
                lax.fori_loop(0, NUM_FEATURES, feat_s, 0)

        zero_acc(0)
        fire_hbm_gathers(0, 0)

        @pl.when(sid < NUM_SUBCORES - 1)
        def _():
            pltpu.make_async_copy(
                tab_hbm.at[pl.ds(sid * STAGE_ROWS, STAGE_ROWS)],
                shared_tab.at[pl.ds(sid * STAGE_ROWS, STAGE_ROWS)],
                ssem,
            ).wait()

        @pl.when(sid == NUM_SUBCORES - 1)
        def _():
            pltpu.make_async_copy(
                tab_hbm.at[pl.ds((NUM_SUBCORES - 1) * STAGE_ROWS, STAGE_TAIL)],
                shared_tab.at[
                    pl.ds((NUM_SUBCORES - 1) * STAGE_ROWS, STAGE_TAIL)],
                ssem,
            ).wait()

        plsc.subcore_barrier()

        zero_acc(1)
        fire_gathers(1, 1)
        zero_acc(2)
        fire_gathers(2, 2)

        def chunk_body(c, _):
            p = c % 3
            drain_gathers(c, p)
            pltpu.async_copy(
                acc2.at[p], out_hbm.at[pl.ds(base + c * CHUNK, CHUNK)],
                osem.at[p],
            )

            @pl.when(c + 3 < N_CHUNKS + 0)
            def _prep_next():
                pltpu.make_async_copy(
                    acc2.at[p], out_hbm.at[pl.ds(base + c * CHUNK, CHUNK)],
                    osem.at[p],
                ).wait()
                zero_acc(p)
                fire_gathers(c + 3, p)

            return 0

        lax.fori_loop(0, N_CHUNKS, chunk_body, 0)

        for c in (N_CHUNKS - 3, N_CHUNKS - 2, N_CHUNKS - 1):
            pltpu.make_async_copy(
                acc2.at[c % 3],
                out_hbm.at[pl.ds(base + c * CHUNK, CHUNK)],
                osem.at[c % 3],
            ).wait()

    return k(xi, tab)


def kernel(x, tables):
    if x.ndim == 1:
        x = x[:, None]
    xi = x.astype(jnp.int32) + NUM_VALUES * jnp.arange(
        NUM_FEATURES, dtype=jnp.int32)[None, :]
    xi = xi.reshape(NUM_WORKERS, N_CHUNKS, CHUNK, NUM_FEATURES)
    xi = xi.transpose(0, 3, 1, 2)
    return _sc_encode(xi, tables.reshape(TAB_ROWS, HIDDEN))
